# baseline (speedup 1.0000x reference)
import functools

import jax
import jax.numpy as jnp
from jax import lax
from jax.experimental import pallas as pl
from jax.experimental.pallas import tpu as pltpu
from jax.experimental.pallas import tpu_sc as plsc

N = 10000
E = 320000
D = 128
L = 2

NC = 2
NS = 16
NW = NC * NS
EPW = E // NW
K = 125
NCH = EPW // K
NSEG = 2
SEG = NCH // NSEG
U = 8
WPR = 624
WREM = N - NS * WPR


def _fill_const_1d(ref, n, val):
    def body(i, c):
        ref[pl.ds(i * 16, 16)] = jnp.full((16,), val, jnp.float32)
        return c
    lax.fori_loop(0, n // 16, body, 0)


def _make_sc_agg(tabs, with_cnt):
    mesh = plsc.VectorSubcoreMesh(
        core_axis_name="c", subcore_axis_name="s",
        num_cores=NC, num_subcores=NS)

    out_type = [jax.ShapeDtypeStruct((NC, N, D), jnp.float32)
                for _ in range(tabs)]
    if with_cnt:
        out_type.append(jax.ShapeDtypeStruct((NC * N,), jnp.float32))

    scratch = dict(
        src_v=pltpu.VMEM((SEG, K), jnp.int32),
        dst_v=pltpu.VMEM((SEG, K), jnp.int32),
        rows0=pltpu.VMEM((K, D), jnp.float32),
        rows1=pltpu.VMEM((K, D), jnp.float32),
        acc_sh=pltpu.VMEM_SHARED((N, D), jnp.float32),
        gsem0=pltpu.SemaphoreType.DMA,
        gsem1=pltpu.SemaphoreType.DMA,
    )
    if with_cnt:
        scratch.update(
            ones_v=pltpu.VMEM((128,), jnp.float32),
            cbuf=pltpu.VMEM((1024,), jnp.float32),
            cnt_sh=pltpu.VMEM_SHARED((N,), jnp.float32),
        )

    @functools.partial(pl.kernel, out_type=out_type, mesh=mesh,
                       scratch_types=scratch)
    def body(*args, src_v, dst_v, rows0, rows1, acc_sh, gsem0, gsem1,
             ones_v=None, cbuf=None, cnt_sh=None):
        tab_hbms = args[:tabs]
        zeros_hbm = args[tabs]
        czero_hbm = args[tabs + 1]
        src_hbm, dst_hbm = args[tabs + 2], args[tabs + 3]
        out_hbms = args[tabs + 4:tabs + 4 + tabs]
        cnt_hbm = args[2 * tabs + 4] if with_cnt else None

        cid = lax.axis_index("c")
        sid = lax.axis_index("s")
        wid = sid * NC + cid

        rows = (rows0, rows1)
        gsem = (gsem0, gsem1)

        if with_cnt:
            _fill_const_1d(ones_v, 128, 1.0)
            @pl.when(sid < 10)
            def _():
                pltpu.sync_copy(czero_hbm, cbuf)
                pltpu.sync_copy(cbuf.at[pl.ds(0, 1000)],
                                cnt_sh.at[pl.ds(sid * 1000, 1000)])

        for t in range(tabs):
            tab = tab_hbms[t]
            do_cnt = with_cnt and t == 0

            def gstart(j, p):
                return pltpu.async_copy(
                    tab.at[src_v.at[j]], rows[p], gsem[p])

            def scat(j, p):
                pltpu.sync_copy(rows[p], acc_sh.at[dst_v.at[j]], add=True)
                if do_cnt:
                    pltpu.sync_copy(ones_v.at[pl.ds(0, K)],
                                    cnt_sh.at[dst_v.at[j]], add=True)

            pltpu.sync_copy(zeros_hbm.at[pl.ds(0, WPR)],
                            acc_sh.at[pl.ds(sid * WPR, WPR)])

            @pl.when(sid == NS - 1)
            def _():
                pltpu.sync_copy(zeros_hbm.at[pl.ds(0, WREM)],
                                acc_sh.at[pl.ds(NS * WPR, WREM)])
            plsc.subcore_barrier()

            def run_chunks(base, n):
                d = [gstart(base, 0), gstart(base + 1, 1)]
                for k in range(n):
                    p = k % 2
                    d[p].wait()
                    scat(base + k, p)
                    if k + 2 < n:
                        d[p] = gstart(base + k + 2, p)
                return 0

            def step(i, c):
                return run_chunks(U * i, U)

            for seg in range(NSEG):
                pltpu.sync_copy(src_hbm.at[wid, seg], src_v)
                pltpu.sync_copy(dst_hbm.at[wid, seg], dst_v)
                lax.fori_loop(0, SEG // U, step, 0)

            plsc.subcore_barrier()
            r0 = sid * WPR
            pltpu.sync_copy(acc_sh.at[pl.ds(r0, WPR)],
                            out_hbms[t].at[cid, pl.ds(r0, WPR)])

            @pl.when(sid == NS - 1)
            def _():
                pltpu.sync_copy(acc_sh.at[pl.ds(NS * WPR, WREM)],
                                out_hbms[t].at[cid, pl.ds(NS * WPR, WREM)])
            if do_cnt:
                @pl.when(sid < 10)
                def _():
                    r0 = sid * 1000
                    pltpu.sync_copy(cnt_sh.at[pl.ds(r0, 1000)],
                                    cbuf.at[pl.ds(0, 1000)])
                    pltpu.sync_copy(cbuf.at[pl.ds(0, 1000)],
                                    cnt_hbm.at[pl.ds(cid * N + r0, 1000)])
            if t + 1 < tabs:
                plsc.subcore_barrier()

    return body


_sc_agg_cache = {}


def _sc_agg(tabs, with_cnt):
    key = (tabs, with_cnt)
    if key not in _sc_agg_cache:
        _sc_agg_cache[key] = _make_sc_agg(tabs, with_cnt)
    return _sc_agg_cache[key]


BN = 1000


def _gates_body(pi_ref, ph_ref, pc_ref, inp_ref, h_ref, wg_ref, bg_ref,
                z_ref, rh_ref, mi_ref):
    cnt = pc_ref[:, 0] + pc_ref[:, 1]
    recip = 1.0 / jnp.maximum(cnt, 1.0)
    mi = (pi_ref[0] + pi_ref[1]) * recip[:, None]
    mh = (ph_ref[0] + ph_ref[1]) * recip[:, None]
    inp = inp_ref[...]
    hh = h_ref[...]
    xcat = jnp.concatenate([mi, inp, mh, hh], axis=1)
    zpre = jnp.dot(xcat, wg_ref[0], preferred_element_type=jnp.float32)
    rpre = jnp.dot(xcat, wg_ref[1], preferred_element_type=jnp.float32)
    z = jax.nn.sigmoid(zpre + bg_ref[0][None, :])
    r = jax.nn.sigmoid(rpre + bg_ref[1][None, :])
    z_ref[...] = z
    rh_ref[...] = r * hh
    mi_ref[...] = mi


def _gates(pi, ph, pc, inp, hh, wg, bg):
    grid = N // BN
    row = lambda n: (n, 0)
    prow = lambda n: (0, n, 0)
    full2 = lambda n: (0, 0)
    full3 = lambda n: (0, 0, 0)
    return pl.pallas_call(
        _gates_body,
        grid=(grid,),
        in_specs=[
            pl.BlockSpec((2, BN, D), prow),
            pl.BlockSpec((2, BN, D), prow),
            pl.BlockSpec((BN, 2), lambda n: (n, 0)),
            pl.BlockSpec((BN, D), row),
            pl.BlockSpec((BN, D), row),
            pl.BlockSpec((2, 4 * D, D), full3),
            pl.BlockSpec((2, D), full2),
        ],
        out_specs=[
            pl.BlockSpec((BN, D), row),
            pl.BlockSpec((BN, D), row),
            pl.BlockSpec((BN, D), row),
        ],
        out_shape=[jax.ShapeDtypeStruct((N, D), jnp.float32)] * 3,
    )(pi, ph, pc, inp, hh, wg, bg)


def _out_body(mi_ref, inp_ref, prh_ref, pc_ref, rh_ref, h_ref, z_ref,
              wh_ref, bh_ref, out_ref):
    cnt = pc_ref[:, 0] + pc_ref[:, 1]
    recip = 1.0 / jnp.maximum(cnt, 1.0)
    mrh = (prh_ref[0] + prh_ref[1]) * recip[:, None]
    xcat = jnp.concatenate([mi_ref[...], inp_ref[...], mrh, rh_ref[...]],
                           axis=1)
    pre = jnp.dot(xcat, wh_ref[...], preferred_element_type=jnp.float32)
    ht = jnp.tanh(pre + bh_ref[0][None, :])
    z = z_ref[...]
    out_ref[...] = z * h_ref[...] + (1.0 - z) * ht


def _out(mi, inp, prh, pc, rh, hh, z, wh, bh):
    grid = N // BN
    row = lambda n: (n, 0)
    prow = lambda n: (0, n, 0)
    return pl.pallas_call(
        _out_body,
        grid=(grid,),
        in_specs=[
            pl.BlockSpec((BN, D), row),
            pl.BlockSpec((BN, D), row),
            pl.BlockSpec((2, BN, D), prow),
            pl.BlockSpec((BN, 2), lambda n: (n, 0)),
            pl.BlockSpec((BN, D), row),
            pl.BlockSpec((BN, D), row),
            pl.BlockSpec((BN, D), row),
            pl.BlockSpec((4 * D, D), lambda n: (0, 0)),
            pl.BlockSpec((1, D), lambda n: (0, 0)),
        ],
        out_specs=pl.BlockSpec((BN, D), row),
        out_shape=jax.ShapeDtypeStruct((N, D), jnp.float32),
    )(mi, inp, prh, pc, rh, hh, z, wh, bh)


def kernel(x, edge_index, h, Wl, Wr, bl):
    src = edge_index[0].reshape(NW, NSEG, SEG, K)
    dst = edge_index[1].reshape(NW, NSEG, SEG, K)
    zeros = jnp.zeros((WPR, D), jnp.float32)
    czero = jnp.zeros((1024,), jnp.float32)

    wg = jnp.stack([
        jnp.concatenate([
            jnp.concatenate([Wl[i, 0], Wr[i, 0], Wl[i, 1], Wr[i, 1]], axis=0)
            [None],
            jnp.concatenate([Wl[i, 2], Wr[i, 2], Wl[i, 3], Wr[i, 3]], axis=0)
            [None],
        ], axis=0) for i in range(L)], axis=0)
    bg = jnp.stack([
        jnp.stack([bl[i, 0] + bl[i, 1], bl[i, 2] + bl[i, 3]], axis=0)
        for i in range(L)], axis=0)
    wh = jnp.stack([
        jnp.concatenate([Wl[i, 4], Wr[i, 4], Wl[i, 5], Wr[i, 5]], axis=0)
        for i in range(L)], axis=0)
    bh = jnp.stack([(bl[i, 4] + bl[i, 5])[None] for i in range(L)], axis=0)

    px, pc = _sc_agg(1, True)(x, zeros, czero, src, dst)
    pc = pc.reshape(NC, N).T
    (ph0,) = _sc_agg(1, False)(h[0], zeros, czero, src, dst)
    (ph1,) = _sc_agg(1, False)(h[1], zeros, czero, src, dst)

    z0, rh0, mi0 = _gates(px, ph0, pc, x, h[0], wg[0], bg[0])
    (prh0,) = _sc_agg(1, False)(rh0, zeros, czero, src, dst)
    out0 = _out(mi0, x, prh0, pc, rh0, h[0], z0, wh[0], bh[0])

    (pinp1,) = _sc_agg(1, False)(out0, zeros, czero, src, dst)
    z1, rh1, mi1 = _gates(pinp1, ph1, pc, out0, h[1], wg[1], bg[1])
    (prh1,) = _sc_agg(1, False)(rh1, zeros, czero, src, dst)
    out1 = _out(mi1, out0, prh1, pc, rh1, h[1], z1, wh[1], bh[1])
    return jnp.stack([out0, out1], axis=0)


# speedup vs baseline: 1.0071x; 1.0071x over previous
import functools

import jax
import jax.numpy as jnp
from jax import lax
from jax.experimental import pallas as pl
from jax.experimental.pallas import tpu as pltpu
from jax.experimental.pallas import tpu_sc as plsc

N = 10000
E = 320000
D = 128
L = 2

NC = 2
NS = 16
NW = NC * NS
EPW = E // NW
K = 125
NCH = EPW // K
NSEG = 2
SEG = NCH // NSEG
U = 8
WOT = 10
WPR = N // WOT


def _fill_const_1d(ref, n, val):
    def body(i, c):
        ref[pl.ds(i * 16, 16)] = jnp.full((16,), val, jnp.float32)
        return c
    lax.fori_loop(0, n // 16, body, 0)


def _make_sc_agg(tabs, with_cnt):
    mesh = plsc.VectorSubcoreMesh(
        core_axis_name="c", subcore_axis_name="s",
        num_cores=NC, num_subcores=NS)

    out_type = [jax.ShapeDtypeStruct((NC, N, D), jnp.float32)
                for _ in range(tabs)]
    if with_cnt:
        out_type.append(jax.ShapeDtypeStruct((NC * N,), jnp.float32))

    scratch = dict(
        src_v=pltpu.VMEM((SEG, K), jnp.int32),
        dst_v=pltpu.VMEM((SEG, K), jnp.int32),
        rows0=pltpu.VMEM((K, D), jnp.float32),
        rows1=pltpu.VMEM((K, D), jnp.float32),
        acc_sh=pltpu.VMEM_SHARED((N, D), jnp.float32),
        gsem0=pltpu.SemaphoreType.DMA,
        gsem1=pltpu.SemaphoreType.DMA,
    )
    if with_cnt:
        scratch.update(
            ones_v=pltpu.VMEM((128,), jnp.float32),
            cbuf=pltpu.VMEM((1024,), jnp.float32),
            cnt_sh=pltpu.VMEM_SHARED((N,), jnp.float32),
        )

    @functools.partial(pl.kernel, out_type=out_type, mesh=mesh,
                       scratch_types=scratch)
    def body(*args, src_v, dst_v, rows0, rows1, acc_sh, gsem0, gsem1,
             ones_v=None, cbuf=None, cnt_sh=None):
        tab_hbms = args[:tabs]
        zeros_hbm = args[tabs]
        czero_hbm = args[tabs + 1]
        src_hbm, dst_hbm = args[tabs + 2], args[tabs + 3]
        out_hbms = args[tabs + 4:tabs + 4 + tabs]
        cnt_hbm = args[2 * tabs + 4] if with_cnt else None

        cid = lax.axis_index("c")
        sid = lax.axis_index("s")
        wid = sid * NC + cid

        rows = (rows0, rows1)
        gsem = (gsem0, gsem1)

        if with_cnt:
            _fill_const_1d(ones_v, 128, 1.0)
            @pl.when(sid < 10)
            def _():
                pltpu.sync_copy(czero_hbm, cbuf)
                pltpu.sync_copy(cbuf.at[pl.ds(0, 1000)],
                                cnt_sh.at[pl.ds(sid * 1000, 1000)])

        for t in range(tabs):
            tab = tab_hbms[t]
            do_cnt = with_cnt and t == 0

            def gstart(j, p):
                return pltpu.async_copy(
                    tab.at[src_v.at[j]], rows[p], gsem[p])

            def scat(j, p):
                pltpu.sync_copy(rows[p], acc_sh.at[dst_v.at[j]], add=True)
                if do_cnt:
                    pltpu.sync_copy(ones_v.at[pl.ds(0, K)],
                                    cnt_sh.at[dst_v.at[j]], add=True)

            @pl.when(sid < WOT)
            def _():
                pltpu.sync_copy(zeros_hbm,
                                acc_sh.at[pl.ds(sid * WPR, WPR)])
            plsc.subcore_barrier()

            def run_chunks(base, n):
                d = [gstart(base, 0), gstart(base + 1, 1)]
                for k in range(n):
                    p = k % 2
                    d[p].wait()
                    scat(base + k, p)
                    if k + 2 < n:
                        d[p] = gstart(base + k + 2, p)
                return 0

            def step(i, c):
                return run_chunks(U * i, U)

            for seg in range(NSEG):
                pltpu.sync_copy(src_hbm.at[wid, seg], src_v)
                pltpu.sync_copy(dst_hbm.at[wid, seg], dst_v)
                lax.fori_loop(0, SEG // U, step, 0)

            plsc.subcore_barrier()
            @pl.when(sid < WOT)
            def _():
                r0 = sid * WPR
                pltpu.sync_copy(acc_sh.at[pl.ds(r0, WPR)],
                                out_hbms[t].at[cid, pl.ds(r0, WPR)])
            if do_cnt:
                @pl.when(sid < 10)
                def _():
                    r0 = sid * 1000
                    pltpu.sync_copy(cnt_sh.at[pl.ds(r0, 1000)],
                                    cbuf.at[pl.ds(0, 1000)])
                    pltpu.sync_copy(cbuf.at[pl.ds(0, 1000)],
                                    cnt_hbm.at[pl.ds(cid * N + r0, 1000)])
            if t + 1 < tabs:
                plsc.subcore_barrier()

    return body


_sc_agg_cache = {}


def _sc_agg(tabs, with_cnt):
    key = (tabs, with_cnt)
    if key not in _sc_agg_cache:
        _sc_agg_cache[key] = _make_sc_agg(tabs, with_cnt)
    return _sc_agg_cache[key]


BN = 1000


def _gates_body(pi_ref, ph_ref, pc_ref, inp_ref, h_ref, wg_ref, bg_ref,
                z_ref, rh_ref, mi_ref):
    cnt = pc_ref[:, 0] + pc_ref[:, 1]
    recip = 1.0 / jnp.maximum(cnt, 1.0)
    mi = (pi_ref[0] + pi_ref[1]) * recip[:, None]
    mh = (ph_ref[0] + ph_ref[1]) * recip[:, None]
    inp = inp_ref[...]
    hh = h_ref[...]
    xcat = jnp.concatenate([mi, inp, mh, hh], axis=1)
    zpre = jnp.dot(xcat, wg_ref[0], preferred_element_type=jnp.float32)
    rpre = jnp.dot(xcat, wg_ref[1], preferred_element_type=jnp.float32)
    z = jax.nn.sigmoid(zpre + bg_ref[0][None, :])
    r = jax.nn.sigmoid(rpre + bg_ref[1][None, :])
    z_ref[...] = z
    rh_ref[...] = r * hh
    mi_ref[...] = mi


def _gates(pi, ph, pc, inp, hh, wg, bg):
    grid = N // BN
    row = lambda n: (n, 0)
    prow = lambda n: (0, n, 0)
    full2 = lambda n: (0, 0)
    full3 = lambda n: (0, 0, 0)
    return pl.pallas_call(
        _gates_body,
        grid=(grid,),
        in_specs=[
            pl.BlockSpec((2, BN, D), prow),
            pl.BlockSpec((2, BN, D), prow),
            pl.BlockSpec((BN, 2), lambda n: (n, 0)),
            pl.BlockSpec((BN, D), row),
            pl.BlockSpec((BN, D), row),
            pl.BlockSpec((2, 4 * D, D), full3),
            pl.BlockSpec((2, D), full2),
        ],
        out_specs=[
            pl.BlockSpec((BN, D), row),
            pl.BlockSpec((BN, D), row),
            pl.BlockSpec((BN, D), row),
        ],
        out_shape=[jax.ShapeDtypeStruct((N, D), jnp.float32)] * 3,
    )(pi, ph, pc, inp, hh, wg, bg)


def _out_body(mi_ref, inp_ref, prh_ref, pc_ref, rh_ref, h_ref, z_ref,
              wh_ref, bh_ref, out_ref):
    cnt = pc_ref[:, 0] + pc_ref[:, 1]
    recip = 1.0 / jnp.maximum(cnt, 1.0)
    mrh = (prh_ref[0] + prh_ref[1]) * recip[:, None]
    xcat = jnp.concatenate([mi_ref[...], inp_ref[...], mrh, rh_ref[...]],
                           axis=1)
    pre = jnp.dot(xcat, wh_ref[...], preferred_element_type=jnp.float32)
    ht = jnp.tanh(pre + bh_ref[0][None, :])
    z = z_ref[...]
    out_ref[...] = z * h_ref[...] + (1.0 - z) * ht


def _out(mi, inp, prh, pc, rh, hh, z, wh, bh):
    grid = N // BN
    row = lambda n: (n, 0)
    prow = lambda n: (0, n, 0)
    return pl.pallas_call(
        _out_body,
        grid=(grid,),
        in_specs=[
            pl.BlockSpec((BN, D), row),
            pl.BlockSpec((BN, D), row),
            pl.BlockSpec((2, BN, D), prow),
            pl.BlockSpec((BN, 2), lambda n: (n, 0)),
            pl.BlockSpec((BN, D), row),
            pl.BlockSpec((BN, D), row),
            pl.BlockSpec((BN, D), row),
            pl.BlockSpec((4 * D, D), lambda n: (0, 0)),
            pl.BlockSpec((1, D), lambda n: (0, 0)),
        ],
        out_specs=pl.BlockSpec((BN, D), row),
        out_shape=jax.ShapeDtypeStruct((N, D), jnp.float32),
    )(mi, inp, prh, pc, rh, hh, z, wh, bh)


def kernel(x, edge_index, h, Wl, Wr, bl):
    src = edge_index[0].reshape(NW, NSEG, SEG, K)
    dst = edge_index[1].reshape(NW, NSEG, SEG, K)
    zeros = jnp.zeros((WPR, D), jnp.float32)
    czero = jnp.zeros((1024,), jnp.float32)

    wg = jnp.stack([
        jnp.concatenate([
            jnp.concatenate([Wl[i, 0], Wr[i, 0], Wl[i, 1], Wr[i, 1]], axis=0)
            [None],
            jnp.concatenate([Wl[i, 2], Wr[i, 2], Wl[i, 3], Wr[i, 3]], axis=0)
            [None],
        ], axis=0) for i in range(L)], axis=0)
    bg = jnp.stack([
        jnp.stack([bl[i, 0] + bl[i, 1], bl[i, 2] + bl[i, 3]], axis=0)
        for i in range(L)], axis=0)
    wh = jnp.stack([
        jnp.concatenate([Wl[i, 4], Wr[i, 4], Wl[i, 5], Wr[i, 5]], axis=0)
        for i in range(L)], axis=0)
    bh = jnp.stack([(bl[i, 4] + bl[i, 5])[None] for i in range(L)], axis=0)

    px, pc = _sc_agg(1, True)(x, zeros, czero, src, dst)
    pc = pc.reshape(NC, N).T
    (ph0,) = _sc_agg(1, False)(h[0], zeros, czero, src, dst)
    (ph1,) = _sc_agg(1, False)(h[1], zeros, czero, src, dst)

    z0, rh0, mi0 = _gates(px, ph0, pc, x, h[0], wg[0], bg[0])
    (prh0,) = _sc_agg(1, False)(rh0, zeros, czero, src, dst)
    out0 = _out(mi0, x, prh0, pc, rh0, h[0], z0, wh[0], bh[0])

    (pinp1,) = _sc_agg(1, False)(out0, zeros, czero, src, dst)
    z1, rh1, mi1 = _gates(pinp1, ph1, pc, out0, h[1], wg[1], bg[1])
    (prh1,) = _sc_agg(1, False)(rh1, zeros, czero, src, dst)
    out1 = _out(mi1, out0, prh1, pc, rh1, h[1], z1, wh[1], bh[1])
    return jnp.stack([out0, out1], axis=0)


# speedup vs baseline: 1.0222x; 1.0150x over previous
import functools

import jax
import jax.numpy as jnp
from jax import lax
from jax.experimental import pallas as pl
from jax.experimental.pallas import tpu as pltpu
from jax.experimental.pallas import tpu_sc as plsc

N = 10000
E = 320000
D = 128
L = 2

NC = 2
NS = 16
NW = NC * NS
EPW = E // NW
K = 125
NCH = EPW // K
NSEG = 2
SEG = NCH // NSEG
U = 10
WOT = 10
WPR = N // WOT


def _fill_const_1d(ref, n, val):
    def body(i, c):
        ref[pl.ds(i * 16, 16)] = jnp.full((16,), val, jnp.float32)
        return c
    lax.fori_loop(0, n // 16, body, 0)


def _make_sc_agg(tabs, with_cnt):
    mesh = plsc.VectorSubcoreMesh(
        core_axis_name="c", subcore_axis_name="s",
        num_cores=NC, num_subcores=NS)

    out_type = [jax.ShapeDtypeStruct((NC, N, D), jnp.float32)
                for _ in range(tabs)]
    if with_cnt:
        out_type.append(jax.ShapeDtypeStruct((NC * N,), jnp.float32))

    scratch = dict(
        src_v=pltpu.VMEM((SEG, K), jnp.int32),
        dst_v=pltpu.VMEM((SEG, K), jnp.int32),
        rows0=pltpu.VMEM((K, D), jnp.float32),
        rows1=pltpu.VMEM((K, D), jnp.float32),
        acc_sh=pltpu.VMEM_SHARED((N, D), jnp.float32),
        gsem0=pltpu.SemaphoreType.DMA,
        gsem1=pltpu.SemaphoreType.DMA,
    )
    if with_cnt:
        scratch.update(
            ones_v=pltpu.VMEM((128,), jnp.float32),
            cbuf=pltpu.VMEM((1024,), jnp.float32),
            cnt_sh=pltpu.VMEM_SHARED((N,), jnp.float32),
        )

    @functools.partial(pl.kernel, out_type=out_type, mesh=mesh,
                       scratch_types=scratch)
    def body(*args, src_v, dst_v, rows0, rows1, acc_sh, gsem0, gsem1,
             ones_v=None, cbuf=None, cnt_sh=None):
        tab_hbms = args[:tabs]
        zeros_hbm = args[tabs]
        czero_hbm = args[tabs + 1]
        src_hbm, dst_hbm = args[tabs + 2], args[tabs + 3]
        out_hbms = args[tabs + 4:tabs + 4 + tabs]
        cnt_hbm = args[2 * tabs + 4] if with_cnt else None

        cid = lax.axis_index("c")
        sid = lax.axis_index("s")
        wid = sid * NC + cid

        rows = (rows0, rows1)
        gsem = (gsem0, gsem1)

        if with_cnt:
            _fill_const_1d(ones_v, 128, 1.0)
            @pl.when(sid < 10)
            def _():
                pltpu.sync_copy(czero_hbm, cbuf)
                pltpu.sync_copy(cbuf.at[pl.ds(0, 1000)],
                                cnt_sh.at[pl.ds(sid * 1000, 1000)])

        for t in range(tabs):
            tab = tab_hbms[t]
            do_cnt = with_cnt and t == 0

            def gstart(j, p):
                return pltpu.async_copy(
                    tab.at[src_v.at[j]], rows[p], gsem[p])

            def scat(j, p):
                pltpu.sync_copy(rows[p], acc_sh.at[dst_v.at[j]], add=True)
                if do_cnt:
                    pltpu.sync_copy(ones_v.at[pl.ds(0, K)],
                                    cnt_sh.at[dst_v.at[j]], add=True)

            @pl.when(sid < WOT)
            def _():
                pltpu.sync_copy(zeros_hbm,
                                acc_sh.at[pl.ds(sid * WPR, WPR)])
            plsc.subcore_barrier()

            def run_chunks(base, n):
                d = [gstart(base, 0), gstart(base + 1, 1)]
                for k in range(n):
                    p = k % 2
                    d[p].wait()
                    scat(base + k, p)
                    if k + 2 < n:
                        d[p] = gstart(base + k + 2, p)
                return 0

            def step(i, c):
                return run_chunks(U * i, U)

            for seg in range(NSEG):
                pltpu.sync_copy(src_hbm.at[wid, seg], src_v)
                pltpu.sync_copy(dst_hbm.at[wid, seg], dst_v)
                lax.fori_loop(0, SEG // U, step, 0)

            plsc.subcore_barrier()
            @pl.when(sid < WOT)
            def _():
                r0 = sid * WPR
                pltpu.sync_copy(acc_sh.at[pl.ds(r0, WPR)],
                                out_hbms[t].at[cid, pl.ds(r0, WPR)])
            if do_cnt:
                @pl.when(sid < 10)
                def _():
                    r0 = sid * 1000
                    pltpu.sync_copy(cnt_sh.at[pl.ds(r0, 1000)],
                                    cbuf.at[pl.ds(0, 1000)])
                    pltpu.sync_copy(cbuf.at[pl.ds(0, 1000)],
                                    cnt_hbm.at[pl.ds(cid * N + r0, 1000)])
            if t + 1 < tabs:
                plsc.subcore_barrier()

    return body


_sc_agg_cache = {}


def _sc_agg(tabs, with_cnt):
    key = (tabs, with_cnt)
    if key not in _sc_agg_cache:
        _sc_agg_cache[key] = _make_sc_agg(tabs, with_cnt)
    return _sc_agg_cache[key]


BN = 1000


def _gates_body(pi_ref, ph_ref, pc_ref, inp_ref, h_ref, wg_ref, bg_ref,
                z_ref, rh_ref, mi_ref):
    cnt = pc_ref[:, 0] + pc_ref[:, 1]
    recip = 1.0 / jnp.maximum(cnt, 1.0)
    mi = (pi_ref[0] + pi_ref[1]) * recip[:, None]
    mh = (ph_ref[0] + ph_ref[1]) * recip[:, None]
    inp = inp_ref[...]
    hh = h_ref[...]
    xcat = jnp.concatenate([mi, inp, mh, hh], axis=1)
    zpre = jnp.dot(xcat, wg_ref[0], preferred_element_type=jnp.float32)
    rpre = jnp.dot(xcat, wg_ref[1], preferred_element_type=jnp.float32)
    z = jax.nn.sigmoid(zpre + bg_ref[0][None, :])
    r = jax.nn.sigmoid(rpre + bg_ref[1][None, :])
    z_ref[...] = z
    rh_ref[...] = r * hh
    mi_ref[...] = mi


def _gates(pi, ph, pc, inp, hh, wg, bg):
    grid = N // BN
    row = lambda n: (n, 0)
    prow = lambda n: (0, n, 0)
    full2 = lambda n: (0, 0)
    full3 = lambda n: (0, 0, 0)
    return pl.pallas_call(
        _gates_body,
        grid=(grid,),
        in_specs=[
            pl.BlockSpec((2, BN, D), prow),
            pl.BlockSpec((2, BN, D), prow),
            pl.BlockSpec((BN, 2), lambda n: (n, 0)),
            pl.BlockSpec((BN, D), row),
            pl.BlockSpec((BN, D), row),
            pl.BlockSpec((2, 4 * D, D), full3),
            pl.BlockSpec((2, D), full2),
        ],
        out_specs=[
            pl.BlockSpec((BN, D), row),
            pl.BlockSpec((BN, D), row),
            pl.BlockSpec((BN, D), row),
        ],
        out_shape=[jax.ShapeDtypeStruct((N, D), jnp.float32)] * 3,
    )(pi, ph, pc, inp, hh, wg, bg)


def _out_body(mi_ref, inp_ref, prh_ref, pc_ref, rh_ref, h_ref, z_ref,
              wh_ref, bh_ref, out_ref):
    cnt = pc_ref[:, 0] + pc_ref[:, 1]
    recip = 1.0 / jnp.maximum(cnt, 1.0)
    mrh = (prh_ref[0] + prh_ref[1]) * recip[:, None]
    xcat = jnp.concatenate([mi_ref[...], inp_ref[...], mrh, rh_ref[...]],
                           axis=1)
    pre = jnp.dot(xcat, wh_ref[...], preferred_element_type=jnp.float32)
    ht = jnp.tanh(pre + bh_ref[0][None, :])
    z = z_ref[...]
    out_ref[...] = z * h_ref[...] + (1.0 - z) * ht


def _out(mi, inp, prh, pc, rh, hh, z, wh, bh):
    grid = N // BN
    row = lambda n: (n, 0)
    prow = lambda n: (0, n, 0)
    return pl.pallas_call(
        _out_body,
        grid=(grid,),
        in_specs=[
            pl.BlockSpec((BN, D), row),
            pl.BlockSpec((BN, D), row),
            pl.BlockSpec((2, BN, D), prow),
            pl.BlockSpec((BN, 2), lambda n: (n, 0)),
            pl.BlockSpec((BN, D), row),
            pl.BlockSpec((BN, D), row),
            pl.BlockSpec((BN, D), row),
            pl.BlockSpec((4 * D, D), lambda n: (0, 0)),
            pl.BlockSpec((1, D), lambda n: (0, 0)),
        ],
        out_specs=pl.BlockSpec((BN, D), row),
        out_shape=jax.ShapeDtypeStruct((N, D), jnp.float32),
    )(mi, inp, prh, pc, rh, hh, z, wh, bh)


def kernel(x, edge_index, h, Wl, Wr, bl):
    src = edge_index[0].reshape(NW, NSEG, SEG, K)
    dst = edge_index[1].reshape(NW, NSEG, SEG, K)
    zeros = jnp.zeros((WPR, D), jnp.float32)
    czero = jnp.zeros((1024,), jnp.float32)

    wg = jnp.stack([
        jnp.concatenate([
            jnp.concatenate([Wl[i, 0], Wr[i, 0], Wl[i, 1], Wr[i, 1]], axis=0)
            [None],
            jnp.concatenate([Wl[i, 2], Wr[i, 2], Wl[i, 3], Wr[i, 3]], axis=0)
            [None],
        ], axis=0) for i in range(L)], axis=0)
    bg = jnp.stack([
        jnp.stack([bl[i, 0] + bl[i, 1], bl[i, 2] + bl[i, 3]], axis=0)
        for i in range(L)], axis=0)
    wh = jnp.stack([
        jnp.concatenate([Wl[i, 4], Wr[i, 4], Wl[i, 5], Wr[i, 5]], axis=0)
        for i in range(L)], axis=0)
    bh = jnp.stack([(bl[i, 4] + bl[i, 5])[None] for i in range(L)], axis=0)

    px, pc = _sc_agg(1, True)(x, zeros, czero, src, dst)
    pc = pc.reshape(NC, N).T
    (ph0,) = _sc_agg(1, False)(h[0], zeros, czero, src, dst)
    (ph1,) = _sc_agg(1, False)(h[1], zeros, czero, src, dst)

    z0, rh0, mi0 = _gates(px, ph0, pc, x, h[0], wg[0], bg[0])
    (prh0,) = _sc_agg(1, False)(rh0, zeros, czero, src, dst)
    out0 = _out(mi0, x, prh0, pc, rh0, h[0], z0, wh[0], bh[0])

    (pinp1,) = _sc_agg(1, False)(out0, zeros, czero, src, dst)
    z1, rh1, mi1 = _gates(pinp1, ph1, pc, out0, h[1], wg[1], bg[1])
    (prh1,) = _sc_agg(1, False)(rh1, zeros, czero, src, dst)
    out1 = _out(mi1, out0, prh1, pc, rh1, h[1], z1, wh[1], bh[1])
    return jnp.stack([out0, out1], axis=0)


# speedup vs baseline: 1.0632x; 1.0401x over previous
import functools

import jax
import jax.numpy as jnp
from jax import lax
from jax.experimental import pallas as pl
from jax.experimental.pallas import tpu as pltpu
from jax.experimental.pallas import tpu_sc as plsc

N = 10000
E = 320000
D = 128
L = 2

NC = 2
NS = 16
NW = NC * NS
EPW = E // NW
K = 125
NCH = EPW // K
NSEG = 2
SEG = NCH // NSEG
U = 20
WOT = 10
WPR = N // WOT


def _fill_const_1d(ref, n, val):
    def body(i, c):
        ref[pl.ds(i * 16, 16)] = jnp.full((16,), val, jnp.float32)
        return c
    lax.fori_loop(0, n // 16, body, 0)


def _make_sc_agg(tabs, with_cnt):
    mesh = plsc.VectorSubcoreMesh(
        core_axis_name="c", subcore_axis_name="s",
        num_cores=NC, num_subcores=NS)

    out_type = [jax.ShapeDtypeStruct((NC, N, D), jnp.float32)
                for _ in range(tabs)]
    if with_cnt:
        out_type.append(jax.ShapeDtypeStruct((NC * N,), jnp.float32))

    scratch = dict(
        src_v=pltpu.VMEM((SEG, K), jnp.int32),
        dst_v=pltpu.VMEM((SEG, K), jnp.int32),
        rows0=pltpu.VMEM((K, D), jnp.float32),
        rows1=pltpu.VMEM((K, D), jnp.float32),
        acc_sh=pltpu.VMEM_SHARED((N, D), jnp.float32),
        gsem0=pltpu.SemaphoreType.DMA,
        gsem1=pltpu.SemaphoreType.DMA,
    )
    if with_cnt:
        scratch.update(
            ones_v=pltpu.VMEM((128,), jnp.float32),
            cbuf=pltpu.VMEM((1024,), jnp.float32),
            cnt_sh=pltpu.VMEM_SHARED((N,), jnp.float32),
        )

    @functools.partial(pl.kernel, out_type=out_type, mesh=mesh,
                       scratch_types=scratch)
    def body(*args, src_v, dst_v, rows0, rows1, acc_sh, gsem0, gsem1,
             ones_v=None, cbuf=None, cnt_sh=None):
        tab_hbms = args[:tabs]
        zeros_hbm = args[tabs]
        czero_hbm = args[tabs + 1]
        src_hbm, dst_hbm = args[tabs + 2], args[tabs + 3]
        out_hbms = args[tabs + 4:tabs + 4 + tabs]
        cnt_hbm = args[2 * tabs + 4] if with_cnt else None

        cid = lax.axis_index("c")
        sid = lax.axis_index("s")
        wid = sid * NC + cid

        rows = (rows0, rows1)
        gsem = (gsem0, gsem1)

        if with_cnt:
            _fill_const_1d(ones_v, 128, 1.0)
            @pl.when(sid < 10)
            def _():
                pltpu.sync_copy(czero_hbm, cbuf)
                pltpu.sync_copy(cbuf.at[pl.ds(0, 1000)],
                                cnt_sh.at[pl.ds(sid * 1000, 1000)])

        for t in range(tabs):
            tab = tab_hbms[t]
            do_cnt = with_cnt and t == 0

            def gstart(j, p):
                return pltpu.async_copy(
                    tab.at[src_v.at[j]], rows[p], gsem[p])

            def scat(j, p):
                pltpu.sync_copy(rows[p], acc_sh.at[dst_v.at[j]], add=True)
                if do_cnt:
                    pltpu.sync_copy(ones_v.at[pl.ds(0, K)],
                                    cnt_sh.at[dst_v.at[j]], add=True)

            @pl.when(sid < WOT)
            def _():
                pltpu.sync_copy(zeros_hbm,
                                acc_sh.at[pl.ds(sid * WPR, WPR)])
            plsc.subcore_barrier()

            def run_chunks(base, n):
                d = [gstart(base, 0), gstart(base + 1, 1)]
                for k in range(n):
                    p = k % 2
                    d[p].wait()
                    scat(base + k, p)
                    if k + 2 < n:
                        d[p] = gstart(base + k + 2, p)
                return 0

            def step(i, c):
                return run_chunks(U * i, U)

            for seg in range(NSEG):
                pltpu.sync_copy(src_hbm.at[wid, seg], src_v)
                pltpu.sync_copy(dst_hbm.at[wid, seg], dst_v)
                lax.fori_loop(0, SEG // U, step, 0)

            plsc.subcore_barrier()
            @pl.when(sid < WOT)
            def _():
                r0 = sid * WPR
                pltpu.sync_copy(acc_sh.at[pl.ds(r0, WPR)],
                                out_hbms[t].at[cid, pl.ds(r0, WPR)])
            if do_cnt:
                @pl.when(sid < 10)
                def _():
                    r0 = sid * 1000
                    pltpu.sync_copy(cnt_sh.at[pl.ds(r0, 1000)],
                                    cbuf.at[pl.ds(0, 1000)])
                    pltpu.sync_copy(cbuf.at[pl.ds(0, 1000)],
                                    cnt_hbm.at[pl.ds(cid * N + r0, 1000)])
            if t + 1 < tabs:
                plsc.subcore_barrier()

    return body


_sc_agg_cache = {}


def _sc_agg(tabs, with_cnt):
    key = (tabs, with_cnt)
    if key not in _sc_agg_cache:
        _sc_agg_cache[key] = _make_sc_agg(tabs, with_cnt)
    return _sc_agg_cache[key]


BN = 1000


def _gates_body(pi_ref, ph_ref, pc_ref, inp_ref, h_ref, wg_ref, bg_ref,
                z_ref, rh_ref, mi_ref):
    cnt = pc_ref[:, 0] + pc_ref[:, 1]
    recip = 1.0 / jnp.maximum(cnt, 1.0)
    mi = (pi_ref[0] + pi_ref[1]) * recip[:, None]
    mh = (ph_ref[0] + ph_ref[1]) * recip[:, None]
    inp = inp_ref[...]
    hh = h_ref[...]
    xcat = jnp.concatenate([mi, inp, mh, hh], axis=1)
    zpre = jnp.dot(xcat, wg_ref[0], preferred_element_type=jnp.float32)
    rpre = jnp.dot(xcat, wg_ref[1], preferred_element_type=jnp.float32)
    z = jax.nn.sigmoid(zpre + bg_ref[0][None, :])
    r = jax.nn.sigmoid(rpre + bg_ref[1][None, :])
    z_ref[...] = z
    rh_ref[...] = r * hh
    mi_ref[...] = mi


def _gates(pi, ph, pc, inp, hh, wg, bg):
    grid = N // BN
    row = lambda n: (n, 0)
    prow = lambda n: (0, n, 0)
    full2 = lambda n: (0, 0)
    full3 = lambda n: (0, 0, 0)
    return pl.pallas_call(
        _gates_body,
        grid=(grid,),
        in_specs=[
            pl.BlockSpec((2, BN, D), prow),
            pl.BlockSpec((2, BN, D), prow),
            pl.BlockSpec((BN, 2), lambda n: (n, 0)),
            pl.BlockSpec((BN, D), row),
            pl.BlockSpec((BN, D), row),
            pl.BlockSpec((2, 4 * D, D), full3),
            pl.BlockSpec((2, D), full2),
        ],
        out_specs=[
            pl.BlockSpec((BN, D), row),
            pl.BlockSpec((BN, D), row),
            pl.BlockSpec((BN, D), row),
        ],
        out_shape=[jax.ShapeDtypeStruct((N, D), jnp.float32)] * 3,
    )(pi, ph, pc, inp, hh, wg, bg)


def _out_body(mi_ref, inp_ref, prh_ref, pc_ref, rh_ref, h_ref, z_ref,
              wh_ref, bh_ref, out_ref):
    cnt = pc_ref[:, 0] + pc_ref[:, 1]
    recip = 1.0 / jnp.maximum(cnt, 1.0)
    mrh = (prh_ref[0] + prh_ref[1]) * recip[:, None]
    xcat = jnp.concatenate([mi_ref[...], inp_ref[...], mrh, rh_ref[...]],
                           axis=1)
    pre = jnp.dot(xcat, wh_ref[...], preferred_element_type=jnp.float32)
    ht = jnp.tanh(pre + bh_ref[0][None, :])
    z = z_ref[...]
    out_ref[...] = z * h_ref[...] + (1.0 - z) * ht


def _out(mi, inp, prh, pc, rh, hh, z, wh, bh):
    grid = N // BN
    row = lambda n: (n, 0)
    prow = lambda n: (0, n, 0)
    return pl.pallas_call(
        _out_body,
        grid=(grid,),
        in_specs=[
            pl.BlockSpec((BN, D), row),
            pl.BlockSpec((BN, D), row),
            pl.BlockSpec((2, BN, D), prow),
            pl.BlockSpec((BN, 2), lambda n: (n, 0)),
            pl.BlockSpec((BN, D), row),
            pl.BlockSpec((BN, D), row),
            pl.BlockSpec((BN, D), row),
            pl.BlockSpec((4 * D, D), lambda n: (0, 0)),
            pl.BlockSpec((1, D), lambda n: (0, 0)),
        ],
        out_specs=pl.BlockSpec((BN, D), row),
        out_shape=jax.ShapeDtypeStruct((N, D), jnp.float32),
    )(mi, inp, prh, pc, rh, hh, z, wh, bh)


def kernel(x, edge_index, h, Wl, Wr, bl):
    src = edge_index[0].reshape(NW, NSEG, SEG, K)
    dst = edge_index[1].reshape(NW, NSEG, SEG, K)
    zeros = jnp.zeros((WPR, D), jnp.float32)
    czero = jnp.zeros((1024,), jnp.float32)

    wg = jnp.stack([
        jnp.concatenate([
            jnp.concatenate([Wl[i, 0], Wr[i, 0], Wl[i, 1], Wr[i, 1]], axis=0)
            [None],
            jnp.concatenate([Wl[i, 2], Wr[i, 2], Wl[i, 3], Wr[i, 3]], axis=0)
            [None],
        ], axis=0) for i in range(L)], axis=0)
    bg = jnp.stack([
        jnp.stack([bl[i, 0] + bl[i, 1], bl[i, 2] + bl[i, 3]], axis=0)
        for i in range(L)], axis=0)
    wh = jnp.stack([
        jnp.concatenate([Wl[i, 4], Wr[i, 4], Wl[i, 5], Wr[i, 5]], axis=0)
        for i in range(L)], axis=0)
    bh = jnp.stack([(bl[i, 4] + bl[i, 5])[None] for i in range(L)], axis=0)

    px, pc = _sc_agg(1, True)(x, zeros, czero, src, dst)
    pc = pc.reshape(NC, N).T
    (ph0,) = _sc_agg(1, False)(h[0], zeros, czero, src, dst)
    (ph1,) = _sc_agg(1, False)(h[1], zeros, czero, src, dst)

    z0, rh0, mi0 = _gates(px, ph0, pc, x, h[0], wg[0], bg[0])
    (prh0,) = _sc_agg(1, False)(rh0, zeros, czero, src, dst)
    out0 = _out(mi0, x, prh0, pc, rh0, h[0], z0, wh[0], bh[0])

    (pinp1,) = _sc_agg(1, False)(out0, zeros, czero, src, dst)
    z1, rh1, mi1 = _gates(pinp1, ph1, pc, out0, h[1], wg[1], bg[1])
    (prh1,) = _sc_agg(1, False)(rh1, zeros, czero, src, dst)
    out1 = _out(mi1, out0, prh1, pc, rh1, h[1], z1, wh[1], bh[1])
    return jnp.stack([out0, out1], axis=0)
